# baseline (speedup 1.0000x reference)
import functools

import jax
import jax.numpy as jnp
from jax import lax
from jax.experimental import pallas as pl
from jax.experimental.pallas import tpu as pltpu
from jax.experimental.pallas import tpu_sc as plsc

N = 512
E = 4096
H = 16
NC = 2
NS = 16
NW = NC * NS
SRC_PER_W = N // NW


def _sc_body(eu_hbm, ev_hbm, ea_hbm, w_hbm, b_hbm, out_hbm,
             eu_v, ev_v, ea_v, w_v, b_v, cnt_v, ptr_v, wptr_v, col_v,
             enc_v, lvl_v, bp_v, q_v, s_v, norm_v):
    wid = lax.axis_index("s") * NC + lax.axis_index("c")
    lane = lax.iota(jnp.int32, H)
    mask0 = lane == 0
    lane01 = jnp.minimum(lane, 1)
    lane03 = jnp.minimum(lane, 3)

    def sload(ref, idx):
        return plsc.load_gather(ref, [jnp.full((H,), idx, jnp.int32)])[0]

    def sstore(ref, idx, val):
        plsc.store_scatter(ref, [jnp.full((H,), idx, jnp.int32)],
                           jnp.full((H,), val), mask=mask0)

    def row16(ref, r):
        return plsc.load_gather(ref, [r * H + lane])

    pltpu.sync_copy(eu_hbm, eu_v)
    pltpu.sync_copy(ev_hbm, ev_v)
    pltpu.sync_copy(ea_hbm, ea_v)
    pltpu.sync_copy(w_hbm, w_v)
    pltpu.sync_copy(b_hbm, b_v)

    zeros16i = jnp.zeros((H,), jnp.int32)
    ones16i = jnp.ones((H,), jnp.int32)

    for k in range(N // H):
        cnt_v[pl.ds(k * H, H)] = zeros16i

    def count_body(eb, _):
        u_vec = plsc.load_gather(eu_v, [eb * H + lane])
        plsc.addupdate_scatter(cnt_v, [u_vec], ones16i)
        return 0
    lax.fori_loop(0, E // H, count_body, 0)

    sstore(ptr_v, 0, jnp.int32(0))

    def prefix_body(k, carry):
        c = plsc.load_gather(cnt_v, [k * H + lane])
        incl = plsc.cumsum(c) + carry
        plsc.store_scatter(ptr_v, [k * H + 1 + lane], incl)
        plsc.store_scatter(wptr_v, [k * H + lane], incl - c)
        return incl[H - 1]
    lax.fori_loop(0, N // H, prefix_body, jnp.int32(0))

    def place_body(e, _):
        u = sload(eu_v, e)
        v = sload(ev_v, e)
        p = sload(wptr_v, u)
        sstore(col_v, p, v)
        sstore(wptr_v, u, p + 1)
        return 0
    lax.fori_loop(0, E, place_body, 0)

    w0 = b_v[:]
    wr0 = w_v[pl.ds(0 * H, H)]
    wr1 = w_v[pl.ds(1 * H, H)]
    wr2 = w_v[pl.ds(2 * H, H)]
    wr3 = w_v[pl.ds(3 * H, H)]

    def enc_body(j, _):
        av = plsc.load_gather(ea_v, [4 * j + lane03])
        row = w0 + av[0] * wr0 + av[1] * wr1 + av[2] * wr2 + av[3] * wr3
        plsc.store_scatter(enc_v, [j * H + lane], row)
        return 0
    lax.fori_loop(0, N, enc_body, 0)

    minus1 = jnp.full((H,), -1, jnp.int32)
    zerof = jnp.zeros((H,), jnp.float32)

    def src_body(s, _):
        i = wid * SRC_PER_W + s

        for k in range(N // H):
            lvl_v[pl.ds(k * H, H)] = minus1

        sstore(lvl_v, i, jnp.int32(0))
        plsc.store_scatter(s_v, [i * H + lane], row16(enc_v, i))
        sstore(q_v, 0, i)

        def bfs_cond(carry):
            head, tail, level, level_end = carry
            return head < tail

        def bfs_body(carry):
            head, tail, level, level_end = carry
            p = sload(q_v, head)
            pe = plsc.load_gather(ptr_v, [p + lane01])
            e0 = pe[0]
            e1 = pe[1]
            sp = row16(s_v, p)

            def edge_body(e, t):
                j = sload(col_v, e)
                l = sload(lvl_v, j)
                new = l == -1
                cand = jnp.logical_or(new, l == level)

                def maybe_claim(t2):
                    do = jnp.logical_or(new, p < sload(bp_v, j))

                    def claim(t3):
                        sstore(bp_v, j, p)
                        plsc.store_scatter(
                            s_v, [j * H + lane], sp + row16(enc_v, j))

                        def push(t4):
                            sstore(lvl_v, j, level)
                            sstore(q_v, t4, j)
                            return t4 + 1
                        return lax.cond(new, push, lambda t4: t4, t3)

                    return lax.cond(do, claim, lambda t3: t3, t2)

                return lax.cond(cand, maybe_claim, lambda t2: t2, t)

            tail = lax.fori_loop(e0, e1, edge_body, tail)
            head = head + 1
            bump = head == level_end
            level = jnp.where(bump, level + 1, level)
            level_end = jnp.where(bump, tail, level_end)
            return head, tail, level, level_end

        lax.while_loop(bfs_cond, bfs_body,
                       (jnp.int32(0), jnp.int32(1), jnp.int32(1),
                        jnp.int32(1)))

        def norm_body(j, _):
            l = sload(lvl_v, j)
            denom = jnp.maximum(l + 1, 1).astype(jnp.float32)
            sj = row16(s_v, j)
            val = jnp.where(l >= 0, sj / jnp.full((H,), denom), zerof)
            plsc.store_scatter(norm_v, [lane * N + j], val)
            return 0
        lax.fori_loop(0, N, norm_body, 0)

        pltpu.sync_copy(norm_v, out_hbm.at[i])
        return 0

    lax.fori_loop(0, SRC_PER_W, src_body, 0)


@jax.jit
def _launch(eu, ev, ea_flat, w_flat, b):
    mesh = plsc.VectorSubcoreMesh(core_axis_name="c", subcore_axis_name="s",
                                  num_cores=NC, num_subcores=NS)
    f = pl.kernel(
        _sc_body,
        out_type=jax.ShapeDtypeStruct((N, H * N), jnp.float32),
        mesh=mesh,
        compiler_params=pltpu.CompilerParams(needs_layout_passes=False),
        scratch_types=[
            pltpu.VMEM((E,), jnp.int32),
            pltpu.VMEM((E,), jnp.int32),
            pltpu.VMEM((4 * N,), jnp.float32),
            pltpu.VMEM((4 * H,), jnp.float32),
            pltpu.VMEM((H,), jnp.float32),
            pltpu.VMEM((N,), jnp.int32),
            pltpu.VMEM((N + 8,), jnp.int32),
            pltpu.VMEM((N,), jnp.int32),
            pltpu.VMEM((E,), jnp.int32),
            pltpu.VMEM((N * H,), jnp.float32),
            pltpu.VMEM((N,), jnp.int32),
            pltpu.VMEM((N,), jnp.int32),
            pltpu.VMEM((N,), jnp.int32),
            pltpu.VMEM((N * H,), jnp.float32),
            pltpu.VMEM((H * N,), jnp.float32),
        ],
    )
    return f(eu, ev, ea_flat, w_flat, b)


def kernel(x, edge_idx, edge_attr, W, b):
    del x
    eu = edge_idx[0]
    ev = edge_idx[1]
    ea_flat = edge_attr[:N].reshape(-1)
    w_flat = W.reshape(-1)
    out = _launch(eu, ev, ea_flat, w_flat, b)
    return jnp.transpose(out.reshape(N, H, N), (1, 0, 2))


# speedup vs baseline: 2.6653x; 2.6653x over previous
import functools

import jax
import jax.numpy as jnp
from jax import lax
from jax.experimental import pallas as pl
from jax.experimental.pallas import tpu as pltpu
from jax.experimental.pallas import tpu_sc as plsc

N = 512
E = 4096
H = 16
NC = 2
NS = 16
NW = NC * NS
SRC_PER_W = N // NW


def _sc_body(eu_hbm, ev_hbm, ea_hbm, w_hbm, b_hbm, out_hbm,
             eu_v, ev_v, ea_v, w_v, b_v, cnt_v, ptr_v, wptr_v, col_v,
             enc_v, lvl_v, jbuf_v, s_v, norm_v, recip_v):
    wid = lax.axis_index("s") * NC + lax.axis_index("c")
    lane = lax.iota(jnp.int32, H)
    mask0 = lane == 0
    lane01 = jnp.minimum(lane, 1)
    lane03 = jnp.minimum(lane, 3)

    def sload(ref, idx):
        return plsc.load_gather(ref, [jnp.full((H,), idx, jnp.int32)])[0]

    def sstore(ref, idx, val):
        plsc.store_scatter(ref, [jnp.full((H,), idx, jnp.int32)],
                           jnp.full((H,), val), mask=mask0)

    def row16(ref, r):
        return plsc.load_gather(ref, [r * H + lane])

    pltpu.sync_copy(eu_hbm, eu_v)
    pltpu.sync_copy(ev_hbm, ev_v)
    pltpu.sync_copy(ea_hbm, ea_v)
    pltpu.sync_copy(w_hbm, w_v)
    pltpu.sync_copy(b_hbm, b_v)

    zeros16i = jnp.zeros((H,), jnp.int32)
    ones16i = jnp.ones((H,), jnp.int32)
    zerof = jnp.zeros((H,), jnp.float32)
    minus1 = jnp.full((H,), -1, jnp.int32)

    for k in range(N // H):
        cnt_v[pl.ds(k * H, H)] = zeros16i

    def count_body(eb, _):
        u_vec = plsc.load_gather(eu_v, [eb * H + lane])
        plsc.addupdate_scatter(cnt_v, [u_vec], ones16i)
        return 0
    lax.fori_loop(0, E // H, count_body, 0)

    sstore(ptr_v, 0, jnp.int32(0))

    def prefix_body(k, carry):
        c = plsc.load_gather(cnt_v, [k * H + lane])
        incl = plsc.cumsum(c) + carry
        plsc.store_scatter(ptr_v, [k * H + 1 + lane], incl)
        plsc.store_scatter(wptr_v, [k * H + lane], incl - c)
        return incl[H - 1]
    lax.fori_loop(0, N // H, prefix_body, jnp.int32(0))

    def place_body(kb, _):
        uv = plsc.load_gather(eu_v, [kb * H + lane])
        vv = plsc.load_gather(ev_v, [kb * H + lane])
        for kk in range(H):
            u = uv[kk]
            v = vv[kk]
            p = sload(wptr_v, u)
            sstore(col_v, p, v)
            sstore(wptr_v, u, p + 1)
        return 0
    lax.fori_loop(0, E // H, place_body, 0)

    bb = b_v[:]
    wr0 = w_v[pl.ds(0 * H, H)]
    wr1 = w_v[pl.ds(1 * H, H)]
    wr2 = w_v[pl.ds(2 * H, H)]
    wr3 = w_v[pl.ds(3 * H, H)]

    def enc_body(j, _):
        av = plsc.load_gather(ea_v, [4 * j + lane03])
        row = bb + av[0] * wr0 + av[1] * wr1 + av[2] * wr2 + av[3] * wr3
        plsc.store_scatter(enc_v, [j * H + lane], row)
        return 0
    lax.fori_loop(0, N, enc_body, 0)

    def recip_body(k, _):
        d = k * H + lane
        r = 1.0 / (d + 1).astype(jnp.float32)
        plsc.store_scatter(recip_v, [d], r)
        return 0
    lax.fori_loop(0, N // H, recip_body, 0)

    def process_node(p, level):
        pe = plsc.load_gather(ptr_v, [p + lane01])
        e0 = pe[0]
        e1 = pe[1]
        nb = (e1 - e0 + (H - 1)) // H
        sp = row16(s_v, p)

        def blk_body(t, cnt):
            base = e0 + t * H
            eidx = jnp.minimum(base + lane, E - 1)
            valid = (base + lane) < e1
            jv = plsc.load_gather(col_v, [eidx])
            jvs = jnp.where(valid, jv, 0)
            lvj = plsc.load_gather(lvl_v, [jvs])
            cand0 = jnp.logical_and(valid, lvj == -1)

            def have_claims(cnt2):
                jbuf_v[pl.ds(0, H)] = jvs

                def claim_cond(c):
                    cand, _ = c
                    return jnp.any(cand)

                def claim_body(c):
                    cand, n = c
                    lp = plsc.all_reduce_ffs(cand)[0]
                    cand = jnp.logical_and(cand, lane != lp)
                    j = sload(jbuf_v, lp)
                    sstore(lvl_v, j, level)
                    plsc.store_scatter(s_v, [j * H + lane],
                                       sp + row16(enc_v, j))
                    return cand, n + 1

                _, n = lax.while_loop(claim_cond, claim_body, (cand0, cnt2))
                return n

            return lax.cond(jnp.any(cand0), have_claims,
                            lambda cnt2: cnt2, cnt)

        return lax.fori_loop(0, nb, blk_body, jnp.int32(0))

    def src_body(s, _):
        i = wid * SRC_PER_W + s

        for k in range(N // H):
            lvl_v[pl.ds(k * H, H)] = minus1

        sstore(lvl_v, i, jnp.int32(0))
        plsc.store_scatter(s_v, [i * H + lane], row16(enc_v, i))

        moved0 = process_node(i, jnp.int32(1))

        def lvl_cond(c):
            level, moved = c
            return moved > 0

        def lvl_body(c):
            level, _ = c

            def scan_body(k, mv):
                lv = plsc.load_gather(lvl_v, [k * H + lane])
                fm = lv == level - 1

                def have_frontier(mv2):
                    def pop_cond(c2):
                        fmm, _ = c2
                        return jnp.any(fmm)

                    def pop_body(c2):
                        fmm, n = c2
                        lp = plsc.all_reduce_ffs(fmm)[0]
                        fmm = jnp.logical_and(fmm, lane != lp)
                        n = n + process_node(k * H + lp, level)
                        return fmm, n

                    _, mv3 = lax.while_loop(pop_cond, pop_body, (fm, mv2))
                    return mv3

                return lax.cond(jnp.any(fm), have_frontier,
                                lambda mv2: mv2, mv)

            moved = lax.fori_loop(0, N // H, scan_body, jnp.int32(0))
            return level + 1, moved

        lax.while_loop(lvl_cond, lvl_body, (jnp.int32(2), moved0))

        def norm_body(j, _):
            l = sload(lvl_v, j)
            r = sload(recip_v, jnp.maximum(l, 0))
            sj = row16(s_v, j)
            val = jnp.where(l >= 0, sj * jnp.full((H,), r), zerof)
            plsc.store_scatter(norm_v, [lane * N + j], val)
            return 0
        lax.fori_loop(0, N, norm_body, 0)

        pltpu.sync_copy(norm_v, out_hbm.at[i])
        return 0

    lax.fori_loop(0, SRC_PER_W, src_body, 0)


@jax.jit
def _launch(eu, ev, ea_flat, w_flat, b):
    mesh = plsc.VectorSubcoreMesh(core_axis_name="c", subcore_axis_name="s",
                                  num_cores=NC, num_subcores=NS)
    f = pl.kernel(
        _sc_body,
        out_type=jax.ShapeDtypeStruct((N, H * N), jnp.float32),
        mesh=mesh,
        compiler_params=pltpu.CompilerParams(needs_layout_passes=False),
        scratch_types=[
            pltpu.VMEM((E,), jnp.int32),
            pltpu.VMEM((E,), jnp.int32),
            pltpu.VMEM((4 * N,), jnp.float32),
            pltpu.VMEM((4 * H,), jnp.float32),
            pltpu.VMEM((H,), jnp.float32),
            pltpu.VMEM((N,), jnp.int32),
            pltpu.VMEM((N + 8,), jnp.int32),
            pltpu.VMEM((N,), jnp.int32),
            pltpu.VMEM((E,), jnp.int32),
            pltpu.VMEM((N * H,), jnp.float32),
            pltpu.VMEM((N,), jnp.int32),
            pltpu.VMEM((H,), jnp.int32),
            pltpu.VMEM((N * H,), jnp.float32),
            pltpu.VMEM((H * N,), jnp.float32),
            pltpu.VMEM((N,), jnp.float32),
        ],
    )
    return f(eu, ev, ea_flat, w_flat, b)


def kernel(x, edge_idx, edge_attr, W, b):
    del x
    eu = edge_idx[0]
    ev = edge_idx[1]
    ea_flat = edge_attr[:N].reshape(-1)
    w_flat = W.reshape(-1)
    out = _launch(eu, ev, ea_flat, w_flat, b)
    return jnp.transpose(out.reshape(N, H, N), (1, 0, 2))


# speedup vs baseline: 2.8992x; 1.0878x over previous
import functools

import jax
import jax.numpy as jnp
from jax import lax
from jax.experimental import pallas as pl
from jax.experimental.pallas import tpu as pltpu
from jax.experimental.pallas import tpu_sc as plsc

N = 512
E = 4096
H = 16
NC = 2
NS = 16
NW = NC * NS
SRC_PER_W = N // NW


def _sc_body(eu_hbm, ev_hbm, ea_hbm, w_hbm, b_hbm, out_hbm,
             eu_v, ev_v, ea_v, w_v, b_v, cnt_v, ptr_v, wptr_v, col_v,
             enc_v, lvl_v, bp_v, s_v, norm_v, recip_v):
    wid = lax.axis_index("s") * NC + lax.axis_index("c")
    lane = lax.iota(jnp.int32, H)
    mask0 = lane == 0
    lane01 = jnp.minimum(lane, 1)
    lane03 = jnp.minimum(lane, 3)

    def sload(ref, idx):
        return plsc.load_gather(ref, [jnp.full((H,), idx, jnp.int32)])[0]

    def sstore(ref, idx, val):
        plsc.store_scatter(ref, [jnp.full((H,), idx, jnp.int32)],
                           jnp.full((H,), val), mask=mask0)

    def row16(ref, r):
        return plsc.load_gather(ref, [r * H + lane])

    pltpu.sync_copy(eu_hbm, eu_v)
    pltpu.sync_copy(ev_hbm, ev_v)
    pltpu.sync_copy(ea_hbm, ea_v)
    pltpu.sync_copy(w_hbm, w_v)
    pltpu.sync_copy(b_hbm, b_v)

    zeros16i = jnp.zeros((H,), jnp.int32)
    ones16i = jnp.ones((H,), jnp.int32)
    zerof = jnp.zeros((H,), jnp.float32)
    minus1 = jnp.full((H,), -1, jnp.int32)

    for k in range(N // H):
        cnt_v[pl.ds(k * H, H)] = zeros16i

    def count_body(eb, _):
        u_vec = plsc.load_gather(eu_v, [eb * H + lane])
        plsc.addupdate_scatter(cnt_v, [u_vec], ones16i)
        return 0
    lax.fori_loop(0, E // H, count_body, 0)

    sstore(ptr_v, 0, jnp.int32(0))

    def prefix_body(k, carry):
        c = plsc.load_gather(cnt_v, [k * H + lane])
        incl = plsc.cumsum(c) + carry
        plsc.store_scatter(ptr_v, [k * H + 1 + lane], incl)
        plsc.store_scatter(wptr_v, [k * H + lane], incl - c)
        return incl[H - 1]
    lax.fori_loop(0, N // H, prefix_body, jnp.int32(0))

    def place_body(kb, _):
        uv = plsc.load_gather(eu_v, [kb * H + lane])
        vv = plsc.load_gather(ev_v, [kb * H + lane])
        for kk in range(H):
            u = uv[kk]
            v = vv[kk]
            p = sload(wptr_v, u)
            sstore(col_v, p, v)
            sstore(wptr_v, u, p + 1)
        return 0
    lax.fori_loop(0, E // H, place_body, 0)

    bb = b_v[:]
    wr0 = w_v[pl.ds(0 * H, H)]
    wr1 = w_v[pl.ds(1 * H, H)]
    wr2 = w_v[pl.ds(2 * H, H)]
    wr3 = w_v[pl.ds(3 * H, H)]

    def enc_body(j, _):
        av = plsc.load_gather(ea_v, [4 * j + lane03])
        row = bb + av[0] * wr0 + av[1] * wr1 + av[2] * wr2 + av[3] * wr3
        plsc.store_scatter(enc_v, [j * H + lane], row)
        return 0
    lax.fori_loop(0, N, enc_body, 0)

    def recip_body(k, _):
        d = k * H + lane
        r = 1.0 / (d + 1).astype(jnp.float32)
        plsc.store_scatter(recip_v, [d], r)
        return 0
    lax.fori_loop(0, N // H, recip_body, 0)

    def process_node(p, level):
        pe = plsc.load_gather(ptr_v, [p + lane01])
        e0 = pe[0]
        e1 = pe[1]
        nb = (e1 - e0 + (H - 1)) // H
        lvec = jnp.full((H,), level)
        pvec = jnp.full((H,), p)

        def blk_body(t, cnt):
            base = e0 + t * H
            eidx = jnp.minimum(base + lane, E - 1)
            valid = (base + lane) < e1
            jv = plsc.load_gather(col_v, [eidx])
            jvs = jnp.where(valid, jv, 0)
            lvj = plsc.load_gather(lvl_v, [jvs])
            cand = jnp.logical_and(valid, lvj == -1)
            plsc.store_scatter(lvl_v, [jvs], lvec, mask=cand)
            plsc.store_scatter(bp_v, [jvs], pvec, mask=cand)
            return cnt + plsc.all_reduce_population_count(cand)[0]

        return lax.fori_loop(0, nb, blk_body, jnp.int32(0))

    def src_body(s, _):
        i = wid * SRC_PER_W + s

        for k in range(N // H):
            lvl_v[pl.ds(k * H, H)] = minus1

        sstore(lvl_v, i, jnp.int32(0))
        plsc.store_scatter(s_v, [i * H + lane], row16(enc_v, i))

        moved0 = process_node(i, jnp.int32(1))

        def lvl_cond(c):
            level, moved = c
            return moved > 0

        def lvl_body(c):
            level, _ = c

            def scan_body(k, mv):
                lv = plsc.load_gather(lvl_v, [k * H + lane])
                fm = lv == level - 1

                def have_frontier(mv2):
                    def pop_cond(c2):
                        fmm, _ = c2
                        return jnp.any(fmm)

                    def pop_body(c2):
                        fmm, n = c2
                        lp = plsc.all_reduce_ffs(fmm)[0]
                        fmm = jnp.logical_and(fmm, lane != lp)
                        n = n + process_node(k * H + lp, level)
                        return fmm, n

                    _, mv3 = lax.while_loop(pop_cond, pop_body, (fm, mv2))
                    return mv3

                return lax.cond(jnp.any(fm), have_frontier,
                                lambda mv2: mv2, mv)

            moved = lax.fori_loop(0, N // H, scan_body, jnp.int32(0))
            return level + 1, moved

        lvlf, _ = lax.while_loop(lvl_cond, lvl_body, (jnp.int32(2), moved0))
        maxlev = lvlf - 2

        for k in range(H * N // H):
            norm_v[pl.ds(k * H, H)] = zerof
        plsc.store_scatter(norm_v, [lane * N + i], row16(enc_v, i))

        def rec_body(l, _):
            rv = jnp.full((H,), sload(recip_v, l))

            def scan_body(k, _):
                lv = plsc.load_gather(lvl_v, [k * H + lane])
                fm = lv == l

                def have(z):
                    def pcond(c2):
                        fmm, _ = c2
                        return jnp.any(fmm)

                    def pbody(c2):
                        fmm, z2 = c2
                        lp = plsc.all_reduce_ffs(fmm)[0]
                        fmm = jnp.logical_and(fmm, lane != lp)
                        j = k * H + lp
                        sj = row16(s_v, sload(bp_v, j)) + row16(enc_v, j)
                        plsc.store_scatter(s_v, [j * H + lane], sj)
                        plsc.store_scatter(norm_v, [lane * N + j], sj * rv)
                        return fmm, z2

                    lax.while_loop(pcond, pbody, (fm, z))
                    return 0

                return lax.cond(jnp.any(fm), have, lambda z: 0, 0)

            lax.fori_loop(0, N // H, scan_body, 0)
            return 0
        lax.fori_loop(1, maxlev + 1, rec_body, 0)

        pltpu.sync_copy(norm_v, out_hbm.at[i])
        return 0

    lax.fori_loop(0, SRC_PER_W, src_body, 0)


@jax.jit
def _launch(eu, ev, ea_flat, w_flat, b):
    mesh = plsc.VectorSubcoreMesh(core_axis_name="c", subcore_axis_name="s",
                                  num_cores=NC, num_subcores=NS)
    f = pl.kernel(
        _sc_body,
        out_type=jax.ShapeDtypeStruct((N, H * N), jnp.float32),
        mesh=mesh,
        compiler_params=pltpu.CompilerParams(needs_layout_passes=False),
        scratch_types=[
            pltpu.VMEM((E,), jnp.int32),
            pltpu.VMEM((E,), jnp.int32),
            pltpu.VMEM((4 * N,), jnp.float32),
            pltpu.VMEM((4 * H,), jnp.float32),
            pltpu.VMEM((H,), jnp.float32),
            pltpu.VMEM((N,), jnp.int32),
            pltpu.VMEM((N + 8,), jnp.int32),
            pltpu.VMEM((N,), jnp.int32),
            pltpu.VMEM((E,), jnp.int32),
            pltpu.VMEM((N * H,), jnp.float32),
            pltpu.VMEM((N,), jnp.int32),
            pltpu.VMEM((N,), jnp.int32),
            pltpu.VMEM((N * H,), jnp.float32),
            pltpu.VMEM((H * N,), jnp.float32),
            pltpu.VMEM((N,), jnp.float32),
        ],
    )
    return f(eu, ev, ea_flat, w_flat, b)


def kernel(x, edge_idx, edge_attr, W, b):
    del x
    eu = edge_idx[0]
    ev = edge_idx[1]
    ea_flat = edge_attr[:N].reshape(-1)
    w_flat = W.reshape(-1)
    out = _launch(eu, ev, ea_flat, w_flat, b)
    return jnp.transpose(out.reshape(N, H, N), (1, 0, 2))
